# baseline (speedup 1.0000x reference)
import jax
import jax.numpy as jnp
from jax.experimental import pallas as pl
from jax.experimental.pallas import tpu as pltpu

_N_LEVELS = 255.0


def _minmax_body(x_ref, so_ref, acc_ref, *, nb):
    i = pl.program_id(0)
    x = x_ref[...]
    bmn = jnp.min(x)
    bmx = jnp.max(x)

    @pl.when(i == 0)
    def _init():
        acc_ref[0] = bmn
        acc_ref[1] = bmx

    @pl.when(i > 0)
    def _acc():
        acc_ref[0] = jnp.minimum(acc_ref[0], bmn)
        acc_ref[1] = jnp.maximum(acc_ref[1], bmx)

    @pl.when(i == nb - 1)
    def _fin():
        mn = acc_ref[0]
        mx = acc_ref[1]
        scale = (mx - mn) / _N_LEVELS
        so_ref[0, 0] = scale
        so_ref[0, 1] = jnp.round(-mn / scale)


def _quant_body(so_ref, x_ref, o_ref):
    scale = so_ref[0, 0]
    offset = so_ref[0, 1]
    inv = 1.0 / scale
    x = x_ref[...]
    xi = jnp.round(x * inv) + offset
    xi = jnp.clip(xi, 0.0, _N_LEVELS)
    o_ref[...] = (xi - offset) * scale


def kernel(x_f):
    rows, cols = x_f.shape
    blk = 256
    nb = rows // blk

    so = pl.pallas_call(
        lambda x_ref, so_ref, acc_ref: _minmax_body(x_ref, so_ref, acc_ref, nb=nb),
        grid=(nb,),
        in_specs=[pl.BlockSpec((blk, cols), lambda i: (i, 0))],
        out_specs=pl.BlockSpec(memory_space=pltpu.SMEM),
        out_shape=jax.ShapeDtypeStruct((1, 2), jnp.float32),
        scratch_shapes=[pltpu.SMEM((2,), jnp.float32)],
    )(x_f)

    x_q = pl.pallas_call(
        _quant_body,
        grid=(nb,),
        in_specs=[
            pl.BlockSpec(memory_space=pltpu.SMEM),
            pl.BlockSpec((blk, cols), lambda i: (i, 0)),
        ],
        out_specs=pl.BlockSpec((blk, cols), lambda i: (i, 0)),
        out_shape=jax.ShapeDtypeStruct((rows, cols), jnp.float32),
    )(so, x_f)
    return x_q
